# Initial kernel scaffold; baseline (speedup 1.0000x reference)
import jax
import jax.numpy as jnp
from jax.experimental import pallas as pl


def kernel(X, edges, W1, b1, Wfc, bfc):
    raise NotImplementedError("write your pallas kernel here")


# speedup vs baseline: 21.6264x; 21.6264x over previous
import functools

import jax
import jax.numpy as jnp
from jax import lax
from jax.experimental import pallas as pl
from jax.experimental.pallas import tpu as pltpu, tpu_sc as plsc

N = 10000
E = 320000
D = 128
H = 128
C = 6

NPAD = 10240
NWORKERS = 32
EPW = E // NWORKERS
CHUNK = 128
NFULL = EPW // CHUNK
TAIL = EPW - NFULL * CHUNK
ROWS_PER_TILE = NPAD // 16


def _fill2d(ref, nrow, ncol16, value):
    def body(i, _):
        r = i // ncol16
        cidx = (i % ncol16) * 16
        ref[r, pl.ds(cidx, 16)] = jnp.full((16,), value, jnp.float32)
        return 0
    lax.fori_loop(0, nrow * ncol16, body, 0)


def _deg_body(dst_hbm, out_hbm, didx, didx_t, ones, zbuf, dacc):
    c = lax.axis_index("c")
    s = lax.axis_index("s")
    wid = s * 2 + c

    _fill2d(ones, CHUNK, 1, 1.0)
    _fill2d(zbuf, CHUNK, 1, 0.0)
    rb = pl.multiple_of(s * ROWS_PER_TILE, 8)
    for k in range(ROWS_PER_TILE // CHUNK):
        pltpu.sync_copy(zbuf, dacc.at[pl.ds(rb + k * CHUNK, CHUNK)])
    plsc.subcore_barrier()

    ebase = wid * EPW

    def body(j, _):
        off = pl.multiple_of(ebase + j * CHUNK, 8)
        pltpu.sync_copy(dst_hbm.at[pl.ds(off, CHUNK)], didx)
        pltpu.sync_copy(ones, dacc.at[didx], add=True)
        return 0

    lax.fori_loop(0, NFULL, body, 0)
    offt = pl.multiple_of(ebase + NFULL * CHUNK, 8)
    pltpu.sync_copy(dst_hbm.at[pl.ds(offt, TAIL)], didx_t)
    pltpu.sync_copy(ones.at[pl.ds(0, TAIL)], dacc.at[didx_t], add=True)

    plsc.subcore_barrier()
    pltpu.sync_copy(dacc.at[pl.ds(rb, ROWS_PER_TILE)],
                    out_hbm.at[c, pl.ds(rb, ROWS_PER_TILE)])


_deg_kernel = functools.partial(
    pl.kernel,
    mesh=plsc.VectorSubcoreMesh(core_axis_name="c", subcore_axis_name="s"),
    out_type=jax.ShapeDtypeStruct((2, NPAD, 16), jnp.float32),
    scratch_types=[
        pltpu.VMEM((CHUNK,), jnp.int32),
        pltpu.VMEM((TAIL,), jnp.int32),
        pltpu.VMEM((CHUNK, 16), jnp.float32),
        pltpu.VMEM((CHUNK, 16), jnp.float32),
        pltpu.VMEM_SHARED((NPAD, 16), jnp.float32),
    ],
)(_deg_body)


def _scatter_body(src_hbm, dst_hbm, g_hbm, out_hbm,
                  sidx, didx, sidx_t, didx_t, rows, acc, sem):
    c = lax.axis_index("c")
    s = lax.axis_index("s")
    wid = s * 2 + c

    _fill2d(rows, CHUNK, 8, 0.0)
    rb = pl.multiple_of(s * ROWS_PER_TILE, 8)
    for k in range(ROWS_PER_TILE // CHUNK):
        pltpu.sync_copy(rows, acc.at[pl.ds(rb + k * CHUNK, CHUNK)])
    plsc.subcore_barrier()

    ebase = wid * EPW

    def body(j, _):
        off = pl.multiple_of(ebase + j * CHUNK, 8)
        pltpu.sync_copy(src_hbm.at[pl.ds(off, CHUNK)], sidx)
        pltpu.sync_copy(dst_hbm.at[pl.ds(off, CHUNK)], didx)
        pltpu.async_copy(g_hbm.at[sidx], rows, sem).wait()
        pltpu.sync_copy(rows, acc.at[didx], add=True)
        return 0

    lax.fori_loop(0, NFULL, body, 0)
    offt = pl.multiple_of(ebase + NFULL * CHUNK, 8)
    pltpu.sync_copy(src_hbm.at[pl.ds(offt, TAIL)], sidx_t)
    pltpu.sync_copy(dst_hbm.at[pl.ds(offt, TAIL)], didx_t)
    pltpu.async_copy(g_hbm.at[sidx_t], rows.at[pl.ds(0, TAIL)], sem).wait()
    pltpu.sync_copy(rows.at[pl.ds(0, TAIL)], acc.at[didx_t], add=True)

    plsc.subcore_barrier()
    pltpu.sync_copy(acc.at[pl.ds(rb, ROWS_PER_TILE)],
                    out_hbm.at[c, pl.ds(rb, ROWS_PER_TILE)])


_scatter_kernel = functools.partial(
    pl.kernel,
    mesh=plsc.VectorSubcoreMesh(core_axis_name="c", subcore_axis_name="s"),
    out_type=jax.ShapeDtypeStruct((2, NPAD, D), jnp.float32),
    scratch_types=[
        pltpu.VMEM((CHUNK,), jnp.int32),
        pltpu.VMEM((CHUNK,), jnp.int32),
        pltpu.VMEM((TAIL,), jnp.int32),
        pltpu.VMEM((TAIL,), jnp.int32),
        pltpu.VMEM((CHUNK, D), jnp.float32),
        pltpu.VMEM_SHARED((NPAD, D), jnp.float32),
        pltpu.SemaphoreType.DMA,
    ],
)(_scatter_body)


def _mm_body(x_ref, w_ref, degp_ref, g_ref, dinv_ref):
    h = jnp.dot(x_ref[...], w_ref[...], preferred_element_type=jnp.float32)
    dsum = degp_ref[0, :N, 0] + degp_ref[1, :N, 0]
    dinv = lax.rsqrt(1.0 + dsum)
    g_ref[...] = h * dinv[:, None]
    dinv_ref[...] = dinv[:, None]


def _ep_body(acc_ref, g_ref, dinv_ref, b1_ref, wfc_ref, bfc_ref, out_ref):
    ssum = acc_ref[0, :N, :] + acc_ref[1, :N, :] + g_ref[...]
    pre = ssum * dinv_ref[...] + b1_ref[...]
    r = jnp.maximum(pre, 0.0)
    out_ref[...] = (jnp.dot(r, wfc_ref[...], preferred_element_type=jnp.float32)
                    + bfc_ref[...])


def kernel(X, edges, W1, b1, Wfc, bfc):
    src = edges[0]
    dst = edges[1]

    degp = _deg_kernel(dst)

    g, dinv = pl.pallas_call(
        _mm_body,
        out_shape=[
            jax.ShapeDtypeStruct((N, H), jnp.float32),
            jax.ShapeDtypeStruct((N, 1), jnp.float32),
        ],
    )(X, W1, degp)

    acc = _scatter_kernel(src, dst, g)

    out = pl.pallas_call(
        _ep_body,
        out_shape=jax.ShapeDtypeStruct((N, C), jnp.float32),
    )(acc, g, dinv, b1.reshape(1, H), Wfc, bfc.reshape(1, C))
    return out
